# baseline (speedup 1.0000x reference)
import jax
import jax.numpy as jnp
from jax import lax
from jax.experimental import pallas as pl
from jax.experimental.pallas import tpu as pltpu
from jax.experimental.pallas import tpu_sc as plsc

NUM_CORES = 2
NUM_SUBCORES = 16
LANES = 16
NUM_WORKERS = NUM_CORES * NUM_SUBCORES

BATCH = 16384
LATENT = 32
PACK = 128 // LATENT
PACKED_ROWS = 1000000 // PACK
B_PER_W = BATCH // NUM_WORKERS
CHUNK = 128
N_CHUNKS = B_PER_W // CHUNK
GROUPS = CHUNK // LANES


def _gmf_body(uidx_hbm, iidx_hbm, utab_hbm, itab_hbm, w_hbm, b_hbm, out_hbm,
              iu_raw, ii_raw, iu_p, ii_p, cb_u, cb_i,
              u_buf0, u_buf1, i_buf0, i_buf1, out_v, wv, bv,
              sem_u0, sem_u1, sem_i0, sem_i1):
    wid = lax.axis_index("s") * NUM_CORES + lax.axis_index("c")
    base = wid * B_PER_W

    pltpu.sync_copy(uidx_hbm.at[pl.ds(base, B_PER_W)], iu_raw)
    pltpu.sync_copy(iidx_hbm.at[pl.ds(base, B_PER_W)], ii_raw)
    pltpu.sync_copy(w_hbm, wv)
    pltpu.sync_copy(b_hbm, bv)

    for c in range(N_CHUNKS):
        for k in range(GROUPS):
            o = c * CHUNK + k * LANES
            su = iu_raw[pl.ds(o, LANES)]
            si = ii_raw[pl.ds(o, LANES)]
            iu_p[c, pl.ds(k * LANES, LANES)] = su >> 2
            ii_p[c, pl.ds(k * LANES, LANES)] = si >> 2
            cb_u[pl.ds(o, LANES)] = (su & 3) << 5
            cb_i[pl.ds(o, LANES)] = (si & 3) << 5

    u_bufs = [u_buf0, u_buf1]
    i_bufs = [i_buf0, i_buf1]
    sems_u = [sem_u0, sem_u1]
    sems_i = [sem_i0, sem_i1]

    def fire(c):
        p = c % 2
        return [
            pltpu.async_copy(utab_hbm.at[iu_p.at[c]], u_bufs[p], sems_u[p]),
            pltpu.async_copy(itab_hbm.at[ii_p.at[c]], i_bufs[p], sems_i[p]),
        ]

    iota = lax.iota(jnp.int32, LANES)
    w_vecs = [wv[pl.ds(d * LANES, LANES)] for d in range(LATENT)]
    b_vec = bv[...]

    cps = fire(0)
    for c in range(N_CHUNKS):
        if c + 1 < N_CHUNKS:
            nxt = fire(c + 1)
        for cp in cps:
            cp.wait()
        ub = u_bufs[c % 2]
        ib = i_bufs[c % 2]

        def group(g, carry, c=c, ub=ub, ib=ib):
            j0 = c * CHUNK + g * LANES
            jvec = j0 + iota
            rvec = g * LANES + iota
            cbu = plsc.load_gather(cb_u, [jvec])
            cbi = plsc.load_gather(cb_i, [jvec])
            accs = [jnp.zeros((LANES,), jnp.float32) for _ in range(4)]
            for d in range(LATENT):
                vu = plsc.load_gather(ub, [rvec, cbu + d])
                vi = plsc.load_gather(ib, [rvec, cbi + d])
                accs[d % 4] = accs[d % 4] + vu * vi * w_vecs[d]
            t = (accs[0] + accs[1]) + (accs[2] + accs[3]) + b_vec
            sig = 1.0 / (1.0 + jnp.exp(-t))
            plsc.store_scatter(out_v, [jvec], sig)
            return carry

        lax.fori_loop(0, GROUPS, group, 0)
        if c + 1 < N_CHUNKS:
            cps = nxt

    pltpu.sync_copy(out_v, out_hbm.at[pl.ds(base, B_PER_W)])


@jax.jit
def _gmf(user_indices, item_indices, utab_p, itab_p, w_cols, b_vec):
    mesh = plsc.VectorSubcoreMesh(core_axis_name="c", subcore_axis_name="s",
                                  num_cores=NUM_CORES, num_subcores=NUM_SUBCORES)
    run = pl.kernel(
        _gmf_body,
        out_type=jax.ShapeDtypeStruct((BATCH,), jnp.float32),
        mesh=mesh,
        compiler_params=pltpu.CompilerParams(needs_layout_passes=False),
        scratch_types=[
            pltpu.VMEM((B_PER_W,), jnp.int32),
            pltpu.VMEM((B_PER_W,), jnp.int32),
            pltpu.VMEM((N_CHUNKS, CHUNK), jnp.int32),
            pltpu.VMEM((N_CHUNKS, CHUNK), jnp.int32),
            pltpu.VMEM((B_PER_W,), jnp.int32),
            pltpu.VMEM((B_PER_W,), jnp.int32),
            pltpu.VMEM((CHUNK, 128), jnp.float32),
            pltpu.VMEM((CHUNK, 128), jnp.float32),
            pltpu.VMEM((CHUNK, 128), jnp.float32),
            pltpu.VMEM((CHUNK, 128), jnp.float32),
            pltpu.VMEM((B_PER_W,), jnp.float32),
            pltpu.VMEM((LATENT * LANES,), jnp.float32),
            pltpu.VMEM((LANES,), jnp.float32),
            pltpu.SemaphoreType.DMA,
            pltpu.SemaphoreType.DMA,
            pltpu.SemaphoreType.DMA,
            pltpu.SemaphoreType.DMA,
        ],
    )
    return run(user_indices, item_indices, utab_p, itab_p, w_cols, b_vec)


def kernel(user_indices, item_indices, user_table, item_table, W, b):
    utab_p = user_table.reshape(PACKED_ROWS, 128)
    itab_p = item_table.reshape(PACKED_ROWS, 128)
    w_cols = jnp.broadcast_to(W.reshape(LATENT, 1), (LATENT, LANES)).reshape(-1)
    b_vec = jnp.broadcast_to(b, (LANES,))
    out = _gmf(user_indices.astype(jnp.int32), item_indices.astype(jnp.int32),
               utab_p, itab_p, w_cols, b_vec)
    return out.reshape(BATCH, 1)


# speedup vs baseline: 3.8124x; 3.8124x over previous
import jax
import jax.numpy as jnp
from jax import lax
from jax.experimental import pallas as pl
from jax.experimental.pallas import tpu as pltpu
from jax.experimental.pallas import tpu_sc as plsc

NUM_CORES = 2
NUM_SUBCORES = 16
LANES = 16
NUM_WORKERS = NUM_CORES * NUM_SUBCORES

BATCH = 16384
LATENT = 32
B_PER_W = BATCH // NUM_WORKERS
GROUPS = B_PER_W // LANES
WIN = 128
QE = 4
QUARTERS = B_PER_W // QE
QBUF = QE * WIN


def _gmf_body(uidx_hbm, iidx_hbm, utabT_hbm, itabT_hbm, w_hbm, b_hbm, out_hbm,
              iu_raw, ii_raw, ub0, ub1, ib0, ib1, colbuf, out_v, wv, bv,
              su0, su1, si0, si1):
    wid = lax.axis_index("s") * NUM_CORES + lax.axis_index("c")
    base = wid * B_PER_W

    pltpu.sync_copy(uidx_hbm.at[pl.ds(base, B_PER_W)],
                    iu_raw.at[pl.ds(0, B_PER_W)])
    pltpu.sync_copy(iidx_hbm.at[pl.ds(base, B_PER_W)],
                    ii_raw.at[pl.ds(0, B_PER_W)])
    pltpu.sync_copy(w_hbm, wv)
    pltpu.sync_copy(b_hbm, bv)

    ubs, ibs = [ub0, ub1], [ib0, ib1]
    sus, sis = [su0, su1], [si0, si1]

    def fire(q_dyn, parity):
        vu = iu_raw[pl.ds(q_dyn * QE, LANES)]
        vi = ii_raw[pl.ds(q_dyn * QE, LANES)]
        for e in range(QE):
            offu = pl.multiple_of((vu[e] >> 7) * WIN, 128)
            offi = pl.multiple_of((vi[e] >> 7) * WIN, 128)
            pltpu.async_copy(utabT_hbm.at[:, pl.ds(offu, WIN)],
                             ubs[parity].at[:, pl.ds(e * WIN, WIN)],
                             sus[parity])
            pltpu.async_copy(itabT_hbm.at[:, pl.ds(offi, WIN)],
                             ibs[parity].at[:, pl.ds(e * WIN, WIN)],
                             sis[parity])

    def drain(parity):
        pltpu.make_async_copy(utabT_hbm.at[:, pl.ds(0, QBUF)],
                              ubs[parity], sus[parity]).wait()
        pltpu.make_async_copy(itabT_hbm.at[:, pl.ds(0, QBUF)],
                              ibs[parity], sis[parity]).wait()

    iota = lax.iota(jnp.int32, LANES)
    iota_hi = iota + LANES
    iota16 = iota * LANES
    w_lo = wv[pl.ds(0, LANES)]
    w_hi = wv[pl.ds(LANES, LANES)]
    b_vec = bv[...]

    fire(0, 0)

    def quarter(q, carry):
        parity = lax.rem(q, 2)

        @pl.when(q < QUARTERS - 1)
        def _():
            lax.cond(parity == 0, lambda: fire(q + 1, 1), lambda: fire(q + 1, 0))

        lax.cond(parity == 0, lambda: drain(0), lambda: drain(1))

        vu = iu_raw[pl.ds(q * QE, LANES)]
        vi = ii_raw[pl.ds(q * QE, LANES)]
        e_base = lax.rem(q, 4) * QE

        def extract(par):
            ub, ib = ubs[par], ibs[par]
            for e in range(QE):
                cu = jnp.full((LANES,), e * WIN, jnp.int32) + (vu[e] & 127)
                ci = jnp.full((LANES,), e * WIN, jnp.int32) + (vi[e] & 127)
                u_lo = plsc.load_gather(ub, [iota, cu])
                u_hi = plsc.load_gather(ub, [iota_hi, cu])
                i_lo = plsc.load_gather(ib, [iota, ci])
                i_hi = plsc.load_gather(ib, [iota_hi, ci])
                p = u_lo * i_lo * w_lo + u_hi * i_hi * w_hi
                plsc.store_scatter(colbuf, [iota16 + (e_base + e)], p)

        lax.cond(parity == 0, lambda: extract(0), lambda: extract(1))

        @pl.when(lax.rem(q, 4) == 3)
        def _():
            acc = colbuf[pl.ds(0, LANES)]
            for r in range(1, LANES):
                acc = acc + colbuf[pl.ds(r * LANES, LANES)]
            t = acc + b_vec
            sig = 1.0 / (1.0 + jnp.exp(-t))
            plsc.store_scatter(out_v, [(q // 4) * LANES + iota], sig)

        return carry

    lax.fori_loop(0, QUARTERS, quarter, 0)

    pltpu.sync_copy(out_v, out_hbm.at[pl.ds(base, B_PER_W)])


@jax.jit
def _gmf(user_indices, item_indices, utabT, itabT, w_flat, b_vec):
    mesh = plsc.VectorSubcoreMesh(core_axis_name="c", subcore_axis_name="s",
                                  num_cores=NUM_CORES, num_subcores=NUM_SUBCORES)
    run = pl.kernel(
        _gmf_body,
        out_type=jax.ShapeDtypeStruct((BATCH,), jnp.float32),
        mesh=mesh,
        compiler_params=pltpu.CompilerParams(needs_layout_passes=False),
        scratch_types=[
            pltpu.VMEM((B_PER_W + LANES,), jnp.int32),
            pltpu.VMEM((B_PER_W + LANES,), jnp.int32),
            pltpu.VMEM((LATENT, QBUF), jnp.float32),
            pltpu.VMEM((LATENT, QBUF), jnp.float32),
            pltpu.VMEM((LATENT, QBUF), jnp.float32),
            pltpu.VMEM((LATENT, QBUF), jnp.float32),
            pltpu.VMEM((LANES * LANES,), jnp.float32),
            pltpu.VMEM((B_PER_W,), jnp.float32),
            pltpu.VMEM((LATENT,), jnp.float32),
            pltpu.VMEM((LANES,), jnp.float32),
            pltpu.SemaphoreType.DMA,
            pltpu.SemaphoreType.DMA,
            pltpu.SemaphoreType.DMA,
            pltpu.SemaphoreType.DMA,
        ],
    )
    return run(user_indices, item_indices, utabT, itabT, w_flat, b_vec)


def kernel(user_indices, item_indices, user_table, item_table, W, b):
    utabT = user_table.T
    itabT = item_table.T
    w_flat = W.reshape(LATENT)
    b_vec = jnp.broadcast_to(b, (LANES,))
    out = _gmf(user_indices.astype(jnp.int32), item_indices.astype(jnp.int32),
               utabT, itabT, w_flat, b_vec)
    return out.reshape(BATCH, 1)


# speedup vs baseline: 4.2327x; 1.1102x over previous
import jax
import jax.numpy as jnp
from jax import lax
from jax.experimental import pallas as pl
from jax.experimental.pallas import tpu as pltpu
from jax.experimental.pallas import tpu_sc as plsc

NUM_CORES = 2
NUM_SUBCORES = 16
LANES = 16
NUM_WORKERS = NUM_CORES * NUM_SUBCORES

BATCH = 16384
LATENT = 32
B_PER_W = BATCH // NUM_WORKERS
GROUPS = B_PER_W // LANES
WIN = 128
QE = 4
QUARTERS = B_PER_W // QE
QBUF = QE * WIN


def _gmf_body(uidx_hbm, iidx_hbm, utabT_hbm, itabT_hbm, w_hbm, b_hbm, out_hbm,
              iu_raw, ii_raw, ub0, ub1, ub2, ib0, ib1, ib2, colbuf, out_v, wv, bv,
              su0, su1, su2, si0, si1, si2):
    wid = lax.axis_index("s") * NUM_CORES + lax.axis_index("c")
    base = wid * B_PER_W

    pltpu.sync_copy(uidx_hbm.at[pl.ds(base, B_PER_W)],
                    iu_raw.at[pl.ds(0, B_PER_W)])
    pltpu.sync_copy(iidx_hbm.at[pl.ds(base, B_PER_W)],
                    ii_raw.at[pl.ds(0, B_PER_W)])
    pltpu.sync_copy(w_hbm, wv)
    pltpu.sync_copy(b_hbm, bv)

    ubs, ibs = [ub0, ub1, ub2], [ib0, ib1, ib2]
    sus, sis = [su0, su1, su2], [si0, si1, si2]

    def fire(q_dyn, parity):
        vu = iu_raw[pl.ds(q_dyn * QE, LANES)]
        vi = ii_raw[pl.ds(q_dyn * QE, LANES)]
        for e in range(QE):
            offu = pl.multiple_of((vu[e] >> 7) * WIN, 128)
            offi = pl.multiple_of((vi[e] >> 7) * WIN, 128)
            pltpu.async_copy(utabT_hbm.at[:, pl.ds(offu, WIN)],
                             ubs[parity].at[:, pl.ds(e * WIN, WIN)],
                             sus[parity])
            pltpu.async_copy(itabT_hbm.at[:, pl.ds(offi, WIN)],
                             ibs[parity].at[:, pl.ds(e * WIN, WIN)],
                             sis[parity])

    def drain(parity):
        pltpu.make_async_copy(utabT_hbm.at[:, pl.ds(0, QBUF)],
                              ubs[parity], sus[parity]).wait()
        pltpu.make_async_copy(itabT_hbm.at[:, pl.ds(0, QBUF)],
                              ibs[parity], sis[parity]).wait()

    iota = lax.iota(jnp.int32, LANES)
    iota_hi = iota + LANES
    iota16 = iota * LANES
    w_lo = wv[pl.ds(0, LANES)]
    w_hi = wv[pl.ds(LANES, LANES)]
    b_vec = bv[...]

    fire(0, 0)
    fire(1, 1)

    def quarter(q, carry):
        parity = lax.rem(q, 3)

        @pl.when(q < QUARTERS - 2)
        def _():
            nxt = lax.rem(q + 2, 3)
            lax.switch(nxt, [lambda: fire(q + 2, 0), lambda: fire(q + 2, 1),
                             lambda: fire(q + 2, 2)])

        lax.switch(parity, [lambda: drain(0), lambda: drain(1),
                            lambda: drain(2)])

        vu = iu_raw[pl.ds(q * QE, LANES)]
        vi = ii_raw[pl.ds(q * QE, LANES)]
        e_base = lax.rem(q, 4) * QE

        def extract(par):
            ub, ib = ubs[par], ibs[par]
            for e in range(QE):
                cu = jnp.full((LANES,), e * WIN, jnp.int32) + (vu[e] & 127)
                ci = jnp.full((LANES,), e * WIN, jnp.int32) + (vi[e] & 127)
                u_lo = plsc.load_gather(ub, [iota, cu])
                u_hi = plsc.load_gather(ub, [iota_hi, cu])
                i_lo = plsc.load_gather(ib, [iota, ci])
                i_hi = plsc.load_gather(ib, [iota_hi, ci])
                p = u_lo * i_lo * w_lo + u_hi * i_hi * w_hi
                plsc.store_scatter(colbuf, [iota16 + (e_base + e)], p)

        lax.switch(parity, [lambda: extract(0), lambda: extract(1),
                            lambda: extract(2)])

        @pl.when(lax.rem(q, 4) == 3)
        def _():
            acc = colbuf[pl.ds(0, LANES)]
            for r in range(1, LANES):
                acc = acc + colbuf[pl.ds(r * LANES, LANES)]
            t = acc + b_vec
            sig = 1.0 / (1.0 + jnp.exp(-t))
            plsc.store_scatter(out_v, [(q // 4) * LANES + iota], sig)

        return carry

    lax.fori_loop(0, QUARTERS, quarter, 0)

    pltpu.sync_copy(out_v, out_hbm.at[pl.ds(base, B_PER_W)])


@jax.jit
def _gmf(user_indices, item_indices, utabT, itabT, w_flat, b_vec):
    mesh = plsc.VectorSubcoreMesh(core_axis_name="c", subcore_axis_name="s",
                                  num_cores=NUM_CORES, num_subcores=NUM_SUBCORES)
    run = pl.kernel(
        _gmf_body,
        out_type=jax.ShapeDtypeStruct((BATCH,), jnp.float32),
        mesh=mesh,
        compiler_params=pltpu.CompilerParams(needs_layout_passes=False),
        scratch_types=[
            pltpu.VMEM((B_PER_W + LANES,), jnp.int32),
            pltpu.VMEM((B_PER_W + LANES,), jnp.int32),
            pltpu.VMEM((LATENT, QBUF), jnp.float32),
            pltpu.VMEM((LATENT, QBUF), jnp.float32),
            pltpu.VMEM((LATENT, QBUF), jnp.float32),
            pltpu.VMEM((LATENT, QBUF), jnp.float32),
            pltpu.VMEM((LATENT, QBUF), jnp.float32),
            pltpu.VMEM((LATENT, QBUF), jnp.float32),
            pltpu.VMEM((LANES * LANES,), jnp.float32),
            pltpu.VMEM((B_PER_W,), jnp.float32),
            pltpu.VMEM((LATENT,), jnp.float32),
            pltpu.VMEM((LANES,), jnp.float32),
            pltpu.SemaphoreType.DMA,
            pltpu.SemaphoreType.DMA,
            pltpu.SemaphoreType.DMA,
            pltpu.SemaphoreType.DMA,
            pltpu.SemaphoreType.DMA,
            pltpu.SemaphoreType.DMA,
        ],
    )
    return run(user_indices, item_indices, utabT, itabT, w_flat, b_vec)


def kernel(user_indices, item_indices, user_table, item_table, W, b):
    utabT = user_table.T
    itabT = item_table.T
    w_flat = W.reshape(LATENT)
    b_vec = jnp.broadcast_to(b, (LANES,))
    out = _gmf(user_indices.astype(jnp.int32), item_indices.astype(jnp.int32),
               utabT, itabT, w_flat, b_vec)
    return out.reshape(BATCH, 1)
